# baseline (speedup 1.0000x reference)
import jax
import jax.numpy as jnp
from jax import lax
from jax.experimental import pallas as pl
from jax.experimental.pallas import tpu as pltpu
from jax.experimental.pallas import tpu_sc as plsc

N = 128
A = 512
C = 1000
L = 16
NG = 4
NR = 8
RH = 128
GW = A // NG
KV = GW // L

MOMENTUM = 0.8


def _body(feat_hbm, lab_hbm, cov_hbm, mean_hbm, amt_hbm,
          cov_out, mean_out, amt_out,
          lab_v, feat_v, cov_blk, mean_blk, amt_w, cnt_blk,
          acc_sum, acc_sq, amt_new_w,
          sem_lab, sem_feat, sem_amt, sem_cov, sem_mean):
    nc = 2
    wid = lax.axis_index("s") * nc + lax.axis_index("c")
    r = wid // NG
    g = wid - r * NG
    rb = jnp.minimum(r * RH, C - RH)
    cb = g * GW

    c_lab = pltpu.async_copy(lab_hbm, lab_v, sem_lab)
    c_feat = pltpu.async_copy(feat_hbm.at[:, pl.ds(cb, GW)], feat_v, sem_feat)
    c_amt = pltpu.async_copy(amt_hbm.at[pl.ds(rb, RH)],
                             amt_w.at[pl.ds(0, RH)], sem_amt)
    c_cov = pltpu.async_copy(cov_hbm.at[pl.ds(rb, RH), pl.ds(cb, GW)],
                             cov_blk, sem_cov)
    c_mean = pltpu.async_copy(mean_hbm.at[pl.ds(rb, RH), pl.ds(cb, GW)],
                              mean_blk, sem_mean)

    zeros = jnp.zeros((L,), jnp.float32)
    ones = jnp.ones((L,), jnp.float32)

    @plsc.parallel_loop(0, RH, unroll=4)
    def _(lr):
        cnt_blk[lr, :] = zeros

    c_lab.wait()
    c_feat.wait()

    def accum(i, _):
        lab16 = lab_v[pl.ds(i * L, L)]
        for j in range(L):
            l = lab16[j]
            lr = l - rb

            @pl.when(jnp.logical_and(l >= rb, lr < RH))
            def _():
                cnt = cnt_blk[lr, :]
                first = cnt[0] == 0.0
                cnt_blk[lr, :] = cnt + ones

                @pl.when(first)
                def _():
                    for k in range(KV):
                        s = pl.ds(k * L, L)
                        f = feat_v[i * L + j, s]
                        acc_sum[lr, s] = f
                        acc_sq[lr, s] = f * f

                @pl.when(jnp.logical_not(first))
                def _():
                    for k in range(KV):
                        s = pl.ds(k * L, L)
                        f = feat_v[i * L + j, s]
                        acc_sum[lr, s] = acc_sum[lr, s] + f
                        acc_sq[lr, s] = acc_sq[lr, s] + f * f
        return 0
    lax.fori_loop(0, N // L, accum, 0)

    c_amt.wait()

    lanes = lax.iota(jnp.int32, L)

    @pl.when(g == 0)
    def _():
        @plsc.parallel_loop(0, RH // L)
        def _(k):
            idx = k * L + lanes
            cnt16 = plsc.load_gather(cnt_blk, [idx, lanes])
            amt_new_w[pl.ds(k * L, L)] = amt_w[pl.ds(k * L, L)] + cnt16

        pltpu.sync_copy(amt_new_w, amt_out.at[pl.ds(rb, RH)])

    c_cov.wait()
    c_mean.wait()

    @plsc.parallel_loop(0, RH, unroll=2)
    def _(lr):
        cnt = cnt_blk[lr, :]

        @pl.when(cnt[0] > 0.0)
        def _():
            amt = amt_w[pl.ds(lr, L)][0]
            w = jnp.maximum(cnt / (cnt + amt), 1.0 - MOMENTUM)
            rc = 1.0 / cnt
            omw = 1.0 - w
            for k in range(KV):
                s = pl.ds(k * L, L)
                ave = acc_sum[lr, s] * rc
                var = acc_sq[lr, s] * rc - ave * ave
                m = mean_blk[lr, s]
                dm = m - ave
                cov_blk[lr, s] = (cov_blk[lr, s] * omw + var * w
                                  + w * omw * dm * dm)
                mean_blk[lr, s] = m * omw + ave * w

    c_cov_o = pltpu.async_copy(cov_blk,
                               cov_out.at[pl.ds(rb, RH), pl.ds(cb, GW)],
                               sem_cov)
    c_mean_o = pltpu.async_copy(mean_blk,
                                mean_out.at[pl.ds(rb, RH), pl.ds(cb, GW)],
                                sem_mean)
    c_cov_o.wait()
    c_mean_o.wait()


_sc_call = pl.kernel(
    _body,
    out_type=(
        jax.ShapeDtypeStruct((C, A), jnp.float32),
        jax.ShapeDtypeStruct((C, A), jnp.float32),
        jax.ShapeDtypeStruct((C,), jnp.float32),
    ),
    mesh=plsc.VectorSubcoreMesh(core_axis_name="c", subcore_axis_name="s"),
    compiler_params=pltpu.CompilerParams(use_tc_tiling_on_sc=True,
                                         needs_layout_passes=False),
    scratch_types=[
        pltpu.VMEM((N,), jnp.int32),
        pltpu.VMEM((N, GW), jnp.float32),
        pltpu.VMEM((RH, GW), jnp.float32),
        pltpu.VMEM((RH, GW), jnp.float32),
        pltpu.VMEM((RH + L,), jnp.float32),
        pltpu.VMEM((RH, L), jnp.float32),
        pltpu.VMEM((RH, GW), jnp.float32),
        pltpu.VMEM((RH, GW), jnp.float32),
        pltpu.VMEM((RH,), jnp.float32),
        pltpu.SemaphoreType.DMA,
        pltpu.SemaphoreType.DMA,
        pltpu.SemaphoreType.DMA,
        pltpu.SemaphoreType.DMA,
        pltpu.SemaphoreType.DMA,
    ],
)


@jax.jit
def kernel(features, labels, covariance, mean, amount):
    return _sc_call(features, labels, covariance, mean, amount)


# speedup vs baseline: 1.2832x; 1.2832x over previous
import jax
import jax.numpy as jnp
from jax import lax
from jax.experimental import pallas as pl
from jax.experimental.pallas import tpu as pltpu
from jax.experimental.pallas import tpu_sc as plsc

N = 128
A = 512
C = 1000
L = 16
NW = 32
W = A // NW
CP = 1024

MOMENTUM = 0.8


def _body(feat_hbm, lab_hbm, cov_hbm, mean_hbm, amt_hbm,
          cov_out, mean_out, amt_out,
          lab_v, feat_v, cov_blk, mean_blk, amt_v, cnt_blk,
          acc_sum, acc_sq, amt_new_v, cov_new_c, mean_new_c,
          sem_lab, sem_feat, sem_amt, sem_cov, sem_mean):
    nc = 2
    wid = lax.axis_index("s") * nc + lax.axis_index("c")
    cb = wid * W

    c_lab = pltpu.async_copy(lab_hbm, lab_v, sem_lab)
    c_feat = pltpu.async_copy(feat_hbm.at[:, pl.ds(cb, W)], feat_v, sem_feat)
    c_amt = pltpu.async_copy(amt_hbm, amt_v.at[pl.ds(0, C)], sem_amt)
    c_cov = pltpu.async_copy(cov_hbm.at[:, pl.ds(cb, W)], cov_blk, sem_cov)
    c_mean = pltpu.async_copy(mean_hbm.at[:, pl.ds(cb, W)], mean_blk, sem_mean)

    zeros = jnp.zeros((L,), jnp.float32)
    ones = jnp.ones((L,), jnp.float32)

    c_lab.wait()

    @plsc.parallel_loop(0, N // L, unroll=2)
    def _(i):
        lab16 = lab_v[pl.ds(i * L, L)]
        for j in range(L):
            l = lab16[j]
            cnt_blk[l, :] = zeros
            acc_sum[l, :] = zeros
            acc_sq[l, :] = zeros

    c_feat.wait()

    def accum(i, _):
        lab16 = lab_v[pl.ds(i * L, L)]
        for j in range(L):
            l = lab16[j]
            f = feat_v[i * L + j, :]
            cnt_blk[l, :] = cnt_blk[l, :] + ones
            acc_sum[l, :] = acc_sum[l, :] + f
            acc_sq[l, :] = acc_sq[l, :] + f * f
        return 0
    lax.fori_loop(0, N // L, accum, 0)

    c_amt.wait()

    rb = wid * 2 * L
    a0 = amt_v[pl.ds(rb, L)]
    a1 = amt_v[pl.ds(rb + L, L)]
    amt_new_v[pl.ds(rb, L)] = a0
    amt_new_v[pl.ds(rb + L, L)] = a1

    lanes = lax.iota(jnp.int32, L)

    @plsc.parallel_loop(0, N // L)
    def _(i):
        lab16 = lab_v[pl.ds(i * L, L)]
        cnt16 = plsc.load_gather(cnt_blk, [lab16, lanes])
        amt16 = plsc.load_gather(amt_v, [lab16])
        plsc.store_scatter(amt_new_v, [lab16], amt16 + cnt16)

    @pl.when(wid < NW - 1)
    def _():
        pltpu.sync_copy(amt_new_v.at[pl.ds(rb, 2 * L)],
                        amt_out.at[pl.ds(rb, 2 * L)])

    @pl.when(wid == NW - 1)
    def _():
        pltpu.sync_copy(amt_new_v.at[pl.ds(C - 8, 8)],
                        amt_out.at[pl.ds(C - 8, 8)])

    c_cov.wait()
    c_mean.wait()

    @plsc.parallel_loop(0, N // L, unroll=2)
    def _(i):
        lab16 = lab_v[pl.ds(i * L, L)]
        for j in range(L):
            l = lab16[j]
            n = i * L + j
            cnt = cnt_blk[l, :]
            amt = amt_v[pl.ds(l, L)][0]
            w = jnp.maximum(cnt / (cnt + amt), 1.0 - MOMENTUM)
            rc = 1.0 / cnt
            ave = acc_sum[l, :] * rc
            var = acc_sq[l, :] * rc - ave * ave
            m = mean_blk[l, :]
            dm = m - ave
            omw = 1.0 - w
            cov_new_c[n, :] = (cov_blk[l, :] * omw + var * w
                               + w * omw * dm * dm)
            mean_new_c[n, :] = m * omw + ave * w

    @plsc.parallel_loop(0, N // L, unroll=2)
    def _(i):
        lab16 = lab_v[pl.ds(i * L, L)]
        for j in range(L):
            l = lab16[j]
            n = i * L + j
            cov_blk[l, :] = cov_new_c[n, :]
            mean_blk[l, :] = mean_new_c[n, :]

    c_cov_o = pltpu.async_copy(cov_blk, cov_out.at[:, pl.ds(cb, W)], sem_cov)
    c_mean_o = pltpu.async_copy(mean_blk, mean_out.at[:, pl.ds(cb, W)],
                                sem_mean)
    c_cov_o.wait()
    c_mean_o.wait()


_sc_call = pl.kernel(
    _body,
    out_type=(
        jax.ShapeDtypeStruct((C, A), jnp.float32),
        jax.ShapeDtypeStruct((C, A), jnp.float32),
        jax.ShapeDtypeStruct((C,), jnp.float32),
    ),
    mesh=plsc.VectorSubcoreMesh(core_axis_name="c", subcore_axis_name="s"),
    compiler_params=pltpu.CompilerParams(use_tc_tiling_on_sc=False,
                                         needs_layout_passes=False),
    scratch_types=[
        pltpu.VMEM((N,), jnp.int32),
        pltpu.VMEM((N, W), jnp.float32),
        pltpu.VMEM((C, W), jnp.float32),
        pltpu.VMEM((C, W), jnp.float32),
        pltpu.VMEM((CP,), jnp.float32),
        pltpu.VMEM((C, W), jnp.float32),
        pltpu.VMEM((C, W), jnp.float32),
        pltpu.VMEM((C, W), jnp.float32),
        pltpu.VMEM((CP,), jnp.float32),
        pltpu.VMEM((N, W), jnp.float32),
        pltpu.VMEM((N, W), jnp.float32),
        pltpu.SemaphoreType.DMA,
        pltpu.SemaphoreType.DMA,
        pltpu.SemaphoreType.DMA,
        pltpu.SemaphoreType.DMA,
        pltpu.SemaphoreType.DMA,
    ],
)


@jax.jit
def kernel(features, labels, covariance, mean, amount):
    return _sc_call(features, labels, covariance, mean, amount)


# speedup vs baseline: 1.3145x; 1.0244x over previous
import jax
import jax.numpy as jnp
from jax import lax
from jax.experimental import pallas as pl
from jax.experimental.pallas import tpu as pltpu
from jax.experimental.pallas import tpu_sc as plsc

N = 128
A = 512
C = 1000
L = 16
NG = 4
NR = 8
RH = 128
GW = A // NG
KV = GW // L

MOMENTUM = 0.8


def _body(feat_hbm, lab_hbm, cov_hbm, mean_hbm, amt_hbm,
          cov_out, mean_out, amt_out,
          lab_v, feat_v, cov_blk, mean_blk, amt_w, cnt_w,
          acc_sum, acc_sq, amt_new_w,
          sem_lab, sem_feat, sem_amt, sem_cov, sem_mean):
    nc = 2
    wid = lax.axis_index("s") * nc + lax.axis_index("c")
    r = wid // NG
    g = wid - r * NG
    rb = jnp.minimum(r * RH, C - RH)
    cb = g * GW

    c_lab = pltpu.async_copy(lab_hbm, lab_v, sem_lab)
    c_feat = pltpu.async_copy(feat_hbm.at[:, pl.ds(cb, GW)], feat_v, sem_feat)
    c_amt = pltpu.async_copy(amt_hbm.at[pl.ds(rb, RH)],
                             amt_w.at[pl.ds(0, RH)], sem_amt)
    c_cov = pltpu.async_copy(cov_hbm.at[pl.ds(rb, RH), pl.ds(cb, GW)],
                             cov_blk, sem_cov)
    c_mean = pltpu.async_copy(mean_hbm.at[pl.ds(rb, RH), pl.ds(cb, GW)],
                              mean_blk, sem_mean)

    zeros = jnp.zeros((L,), jnp.float32)
    ones = jnp.ones((L,), jnp.float32)
    zeros_i = jnp.zeros((L,), jnp.int32)
    lanes = lax.iota(jnp.int32, L)

    @plsc.parallel_loop(0, (RH + L) // L, unroll=2)
    def _(i):
        cnt_w[pl.ds(i * L, L)] = zeros

    @plsc.parallel_loop(0, RH, unroll=4)
    def _(row):
        for k in range(KV):
            s = pl.ds(k * L, L)
            acc_sum[row, s] = zeros
            acc_sq[row, s] = zeros

    c_lab.wait()
    c_feat.wait()

    @plsc.parallel_loop(0, N // L, unroll=2)
    def _(i):
        lab16 = lab_v[pl.ds(i * L, L)]
        lr16 = lab16 - rb
        m = jnp.logical_and(lr16 >= 0, lr16 < RH)
        idx = jnp.where(m, lr16, 0)
        plsc.addupdate_scatter(cnt_w, [idx], ones, mask=m)
        for j in range(L):
            l = lab16[j]
            lr = l - rb

            @pl.when(jnp.logical_and(l >= rb, lr < RH))
            def _():
                row16 = lr + zeros_i
                n = i * L + j
                for k in range(KV):
                    s = pl.ds(k * L, L)
                    f = feat_v[n, s]
                    col = k * L + lanes
                    plsc.addupdate_scatter(acc_sum, [row16, col], f)
                    plsc.addupdate_scatter(acc_sq, [row16, col], f * f)

    c_amt.wait()

    @pl.when(g == 0)
    def _():
        @plsc.parallel_loop(0, RH // L)
        def _(k):
            s = pl.ds(k * L, L)
            amt_new_w[s] = amt_w[s] + cnt_w[s]

        pltpu.sync_copy(amt_new_w, amt_out.at[pl.ds(rb, RH)])

    c_cov.wait()
    c_mean.wait()

    @plsc.parallel_loop(0, RH, unroll=2)
    def _(lr):
        cnt = cnt_w[pl.ds(lr, L)][0]

        @pl.when(cnt > 0.0)
        def _():
            cntv = cnt + zeros
            amtv = amt_w[pl.ds(lr, L)][0] + zeros
            w = jnp.maximum(cntv / (cntv + amtv), 1.0 - MOMENTUM)
            rc = ones / cntv
            omw = 1.0 - w
            for k in range(KV):
                s = pl.ds(k * L, L)
                ave = acc_sum[lr, s] * rc
                var = acc_sq[lr, s] * rc - ave * ave
                mn = mean_blk[lr, s]
                dm = mn - ave
                cov_blk[lr, s] = (cov_blk[lr, s] * omw + var * w
                                  + w * omw * dm * dm)
                mean_blk[lr, s] = mn * omw + ave * w

    c_cov_o = pltpu.async_copy(cov_blk,
                               cov_out.at[pl.ds(rb, RH), pl.ds(cb, GW)],
                               sem_cov)
    c_mean_o = pltpu.async_copy(mean_blk,
                                mean_out.at[pl.ds(rb, RH), pl.ds(cb, GW)],
                                sem_mean)
    c_cov_o.wait()
    c_mean_o.wait()


_sc_call = pl.kernel(
    _body,
    out_type=(
        jax.ShapeDtypeStruct((C, A), jnp.float32),
        jax.ShapeDtypeStruct((C, A), jnp.float32),
        jax.ShapeDtypeStruct((C,), jnp.float32),
    ),
    mesh=plsc.VectorSubcoreMesh(core_axis_name="c", subcore_axis_name="s"),
    compiler_params=pltpu.CompilerParams(use_tc_tiling_on_sc=True,
                                         needs_layout_passes=False),
    scratch_types=[
        pltpu.VMEM((N,), jnp.int32),
        pltpu.VMEM((N, GW), jnp.float32),
        pltpu.VMEM((RH, GW), jnp.float32),
        pltpu.VMEM((RH, GW), jnp.float32),
        pltpu.VMEM((RH + L,), jnp.float32),
        pltpu.VMEM((RH + L,), jnp.float32),
        pltpu.VMEM((RH, GW), jnp.float32),
        pltpu.VMEM((RH, GW), jnp.float32),
        pltpu.VMEM((RH,), jnp.float32),
        pltpu.SemaphoreType.DMA,
        pltpu.SemaphoreType.DMA,
        pltpu.SemaphoreType.DMA,
        pltpu.SemaphoreType.DMA,
        pltpu.SemaphoreType.DMA,
    ],
)


@jax.jit
def kernel(features, labels, covariance, mean, amount):
    return _sc_call(features, labels, covariance, mean, amount)


# speedup vs baseline: 1.3624x; 1.0365x over previous
import jax
import jax.numpy as jnp
from jax import lax
from jax.experimental import pallas as pl
from jax.experimental.pallas import tpu as pltpu
from jax.experimental.pallas import tpu_sc as plsc

N = 128
A = 512
C = 1000
L = 16
NG = 4
NR = 8
RH = 128
GW = A // NG
KV = GW // L

MOMENTUM = 0.8


def _body(feat_hbm, lab_hbm, cov_hbm, mean_hbm, amt_hbm,
          cov_out, mean_out, amt_out,
          lab_v, feat_v, cov_blk, mean_blk, amt_w, cnt_w,
          acc_sum, acc_sq, amt_new_w,
          sem_lab, sem_feat, sem_amt, sem_cov, sem_mean):
    nc = 2
    wid = lax.axis_index("s") * nc + lax.axis_index("c")
    r = wid // NG
    g = wid - r * NG
    rb = jnp.minimum(r * RH, C - RH)
    cb = g * GW

    c_lab = pltpu.async_copy(lab_hbm, lab_v, sem_lab)
    c_feat = pltpu.async_copy(feat_hbm.at[:, pl.ds(cb, GW)], feat_v, sem_feat)
    c_amt = pltpu.async_copy(amt_hbm.at[pl.ds(rb, RH)],
                             amt_w.at[pl.ds(0, RH)], sem_amt)
    c_cov = pltpu.async_copy(cov_hbm.at[pl.ds(rb, RH), pl.ds(cb, GW)],
                             cov_blk, sem_cov)
    c_mean = pltpu.async_copy(mean_hbm.at[pl.ds(rb, RH), pl.ds(cb, GW)],
                              mean_blk, sem_mean)

    zeros = jnp.zeros((L,), jnp.float32)
    ones = jnp.ones((L,), jnp.float32)
    zeros_i = jnp.zeros((L,), jnp.int32)
    lanes = lax.iota(jnp.int32, L)

    @plsc.parallel_loop(0, (RH + L) // L, unroll=2)
    def _(i):
        cnt_w[pl.ds(i * L, L)] = zeros

    @plsc.parallel_loop(0, RH, unroll=4)
    def _(row):
        for k in range(KV):
            s = pl.ds(k * L, L)
            acc_sum[row, s] = zeros
            acc_sq[row, s] = zeros

    c_lab.wait()
    c_feat.wait()

    @plsc.parallel_loop(0, N // L, unroll=2)
    def _(i):
        lab16 = lab_v[pl.ds(i * L, L)]
        lr16 = lab16 - rb
        m = jnp.logical_and(lr16 >= 0, lr16 < RH)
        idx = jnp.where(m, lr16, 0)
        plsc.addupdate_scatter(cnt_w, [idx], ones, mask=m)
        for j in range(L):
            row16 = lr16[j] + zeros_i
            msk = jnp.logical_and(row16 >= 0, row16 < RH)
            rowc = jnp.where(msk, row16, 0)
            n = i * L + j
            for k in range(KV):
                s = pl.ds(k * L, L)
                f = feat_v[n, s]
                col = k * L + lanes
                plsc.addupdate_scatter(acc_sum, [rowc, col], f, mask=msk)
                plsc.addupdate_scatter(acc_sq, [rowc, col], f * f, mask=msk)

    c_amt.wait()

    @pl.when(g == 0)
    def _():
        @plsc.parallel_loop(0, RH // L)
        def _(k):
            s = pl.ds(k * L, L)
            amt_new_w[s] = amt_w[s] + cnt_w[s]

        pltpu.sync_copy(amt_new_w, amt_out.at[pl.ds(rb, RH)])

    c_cov.wait()
    c_mean.wait()

    H = RH // 2

    def update(lr):
        cnt = cnt_w[pl.ds(lr, L)][0]

        @pl.when(cnt > 0.0)
        def _():
            cntv = cnt + zeros
            amtv = amt_w[pl.ds(lr, L)][0] + zeros
            w = jnp.maximum(cntv / (cntv + amtv), 1.0 - MOMENTUM)
            rc = ones / cntv
            omw = 1.0 - w
            for k in range(KV):
                s = pl.ds(k * L, L)
                ave = acc_sum[lr, s] * rc
                var = acc_sq[lr, s] * rc - ave * ave
                mn = mean_blk[lr, s]
                dm = mn - ave
                cov_blk[lr, s] = (cov_blk[lr, s] * omw + var * w
                                  + w * omw * dm * dm)
                mean_blk[lr, s] = mn * omw + ave * w

    plsc.parallel_loop(0, H, unroll=2)(update)

    c_cov_o1 = pltpu.async_copy(
        cov_blk.at[pl.ds(0, H)],
        cov_out.at[pl.ds(rb, H), pl.ds(cb, GW)], sem_cov)
    c_mean_o1 = pltpu.async_copy(
        mean_blk.at[pl.ds(0, H)],
        mean_out.at[pl.ds(rb, H), pl.ds(cb, GW)], sem_mean)

    plsc.parallel_loop(H, RH, unroll=2)(update)

    c_cov_o2 = pltpu.async_copy(
        cov_blk.at[pl.ds(H, H)],
        cov_out.at[pl.ds(rb + H, H), pl.ds(cb, GW)], sem_cov)
    c_mean_o2 = pltpu.async_copy(
        mean_blk.at[pl.ds(H, H)],
        mean_out.at[pl.ds(rb + H, H), pl.ds(cb, GW)], sem_mean)
    c_cov_o1.wait()
    c_mean_o1.wait()
    c_cov_o2.wait()
    c_mean_o2.wait()


_sc_call = pl.kernel(
    _body,
    out_type=(
        jax.ShapeDtypeStruct((C, A), jnp.float32),
        jax.ShapeDtypeStruct((C, A), jnp.float32),
        jax.ShapeDtypeStruct((C,), jnp.float32),
    ),
    mesh=plsc.VectorSubcoreMesh(core_axis_name="c", subcore_axis_name="s"),
    compiler_params=pltpu.CompilerParams(use_tc_tiling_on_sc=True,
                                         needs_layout_passes=False),
    scratch_types=[
        pltpu.VMEM((N,), jnp.int32),
        pltpu.VMEM((N, GW), jnp.float32),
        pltpu.VMEM((RH, GW), jnp.float32),
        pltpu.VMEM((RH, GW), jnp.float32),
        pltpu.VMEM((RH + L,), jnp.float32),
        pltpu.VMEM((RH + L,), jnp.float32),
        pltpu.VMEM((RH, GW), jnp.float32),
        pltpu.VMEM((RH, GW), jnp.float32),
        pltpu.VMEM((RH,), jnp.float32),
        pltpu.SemaphoreType.DMA,
        pltpu.SemaphoreType.DMA,
        pltpu.SemaphoreType.DMA,
        pltpu.SemaphoreType.DMA,
        pltpu.SemaphoreType.DMA,
    ],
)


@jax.jit
def kernel(features, labels, covariance, mean, amount):
    return _sc_call(features, labels, covariance, mean, amount)
